# baseline (speedup 1.0000x reference)
import functools

import jax
import jax.numpy as jnp
from jax import lax
from jax.experimental import pallas as pl
from jax.experimental.pallas import tpu as pltpu
from jax.experimental.pallas import tpu_sc as plsc

_NUM_PONDS = 10
_CAPACITY = 10000


def _idx_body(x_ref, pond_ref, out_ref):
    s = jnp.sum(x_ref[...], axis=-1)
    flavor = jnp.abs(s).astype(jnp.int32) % _CAPACITY
    out_ref[...] = pond_ref[...] * _CAPACITY + flavor


def _compute_indices(x, pond):
    n, d = x.shape
    rows = 1024
    grid = n // rows
    return pl.pallas_call(
        _idx_body,
        grid=(grid,),
        in_specs=[
            pl.BlockSpec((rows, d), lambda i: (i, 0)),
            pl.BlockSpec((rows,), lambda i: (i,)),
        ],
        out_specs=pl.BlockSpec((rows,), lambda i: (i,)),
        out_shape=jax.ShapeDtypeStruct((n,), jnp.int32),
    )(x, pond)


@functools.cache
def _make_gather(v, d, n):
    info = plsc.get_sparse_core_info()
    nw = info.num_cores * info.num_subcores
    rows_per_w = n // nw
    ch = 32
    n_chunks = rows_per_w // ch

    mesh = plsc.VectorSubcoreMesh(core_axis_name="c", subcore_axis_name="s")

    @functools.partial(
        pl.kernel,
        mesh=mesh,
        out_type=jax.ShapeDtypeStruct((n, d), jnp.float32),
        scratch_types=[
            pltpu.VMEM((rows_per_w,), jnp.int32),
            pltpu.VMEM((ch, d), jnp.float32),
            pltpu.VMEM((ch, d), jnp.float32),
            pltpu.SemaphoreType.DMA,
            pltpu.SemaphoreType.DMA,
        ],
    )
    def gather(table_hbm, idx_hbm, out_hbm, idx_v, buf0, buf1, sem0, sem1):
        wid = lax.axis_index("s") * info.num_cores + lax.axis_index("c")
        base = wid * rows_per_w
        pltpu.sync_copy(idx_hbm.at[pl.ds(base, rows_per_w)], idx_v)
        bufs = (buf0, buf1)
        sems = (sem0, sem1)
        for c in range(n_chunks):
            buf = bufs[c % 2]
            sem = sems[c % 2]
            cp = pltpu.async_copy(
                table_hbm.at[idx_v.at[pl.ds(c * ch, ch)]], buf, sem)
            cp.wait()
            pltpu.sync_copy(buf, out_hbm.at[pl.ds(base + c * ch, ch)])

    return gather


def kernel(context_vector, pond_assignments, tables):
    b, s, d = context_vector.shape
    n = b * s
    x = context_vector.reshape(n, d)
    pond = pond_assignments.reshape(n).astype(jnp.int32)
    idx = _compute_indices(x, pond)
    table_flat = tables.reshape(_NUM_PONDS * _CAPACITY, d)
    out = _make_gather(_NUM_PONDS * _CAPACITY, d, n)(table_flat, idx)
    return out.reshape(b, s, d)


# speedup vs baseline: 1.0613x; 1.0613x over previous
import functools

import jax
import jax.numpy as jnp
from jax import lax
from jax.experimental import pallas as pl
from jax.experimental.pallas import tpu as pltpu
from jax.experimental.pallas import tpu_sc as plsc

_NUM_PONDS = 10
_CAPACITY = 10000


def _idx_body(x_ref, pond_ref, out_ref):
    s = jnp.sum(x_ref[...], axis=-1)
    flavor = jnp.abs(s).astype(jnp.int32) % _CAPACITY
    out_ref[...] = pond_ref[...] * _CAPACITY + flavor


def _compute_indices(x, pond):
    n, d = x.shape
    rows = 1024
    grid = n // rows
    return pl.pallas_call(
        _idx_body,
        grid=(grid,),
        in_specs=[
            pl.BlockSpec((rows, d), lambda i: (i, 0)),
            pl.BlockSpec((rows,), lambda i: (i,)),
        ],
        out_specs=pl.BlockSpec((rows,), lambda i: (i,)),
        out_shape=jax.ShapeDtypeStruct((n,), jnp.int32),
    )(x, pond)


@functools.cache
def _make_gather(v, d, n):
    info = plsc.get_sparse_core_info()
    nw = info.num_cores * info.num_subcores
    rows_per_w = n // nw
    ch = 32
    nbuf = 3
    n_chunks = rows_per_w // ch

    mesh = plsc.VectorSubcoreMesh(core_axis_name="c", subcore_axis_name="s")

    @functools.partial(
        pl.kernel,
        mesh=mesh,
        out_type=jax.ShapeDtypeStruct((n, d), jnp.float32),
        scratch_types=[
            pltpu.VMEM((rows_per_w,), jnp.int32),
            *[pltpu.VMEM((ch, d), jnp.float32) for _ in range(nbuf)],
            *[pltpu.SemaphoreType.DMA for _ in range(2 * nbuf)],
        ],
    )
    def gather(table_hbm, idx_hbm, out_hbm, idx_v, *scratch):
        bufs = scratch[:nbuf]
        gsem = scratch[nbuf:2 * nbuf]
        osem = scratch[2 * nbuf:]
        wid = lax.axis_index("s") * info.num_cores + lax.axis_index("c")
        base = wid * rows_per_w
        pltpu.sync_copy(idx_hbm.at[pl.ds(base, rows_per_w)], idx_v)

        def start_gather(c, b):
            return pltpu.async_copy(
                table_hbm.at[idx_v.at[pl.ds(c * ch, ch)]], bufs[b], gsem[b])

        gcp = [None] * nbuf
        ocp = [None] * nbuf
        for c in range(nbuf):
            gcp[c] = start_gather(c, c)
        for c in range(n_chunks):
            b = c % nbuf
            gcp[b].wait()
            ocp[b] = pltpu.async_copy(
                bufs[b], out_hbm.at[pl.ds(base + c * ch, ch)], osem[b])
            nxt = c + nbuf
            if nxt < n_chunks:
                ocp[b].wait()
                gcp[b] = start_gather(nxt, b)
        for c in range(n_chunks - nbuf, n_chunks):
            ocp[c % nbuf].wait()

    return gather


def kernel(context_vector, pond_assignments, tables):
    b, s, d = context_vector.shape
    n = b * s
    x = context_vector.reshape(n, d)
    pond = pond_assignments.reshape(n).astype(jnp.int32)
    idx = _compute_indices(x, pond)
    table_flat = tables.reshape(_NUM_PONDS * _CAPACITY, d)
    out = _make_gather(_NUM_PONDS * _CAPACITY, d, n)(table_flat, idx)
    return out.reshape(b, s, d)
